# baseline (speedup 1.0000x reference)
import functools

import jax
import jax.numpy as jnp
from jax import lax
from jax.experimental import pallas as pl
from jax.experimental.pallas import tpu as pltpu
from jax.experimental.pallas import tpu_sc as plsc

N = 10000
E = 160000
D = 256
H = D // 2
P = 3
NC = 2
NS = 16
EPW = E // NS
CH = 128
NCH = 80
EPAD = NCH * CH - EPW
ACCR = N + 16
RPS = 624
RREM = N - NS * RPS
RT = 1000
PATH_LIST = ((1,), (1, 2), (1, 2, 3))

_f32 = jnp.float32


def _sc_body(xflat_hbm, s0, d0, s1, d1, s2, d2, out_hbm,
             sidx0, sidx1, didx0, didx1, didx2, didx3, rows0, rows1,
             acc, gs0, gs1, ss0, ss1):
    c = lax.axis_index("c")
    s = lax.axis_index("s")
    ebase = s * (NCH * CH)
    row0 = s * RPS
    coff = c * N
    srcs = (s0, s1, s2)
    dsts = (d0, d1, d2)
    sidx_ = (sidx0, sidx1)
    didx_ = (didx0, didx1, didx2, didx3)
    rows_ = (rows0, rows1)
    gsem_ = (gs0, gs1)
    ssem_ = (ss0, ss1)

    def load_idx(i, g, sbuf, dbuf):
        off = ebase + g * CH
        pltpu.sync_copy(srcs[i].at[pl.ds(off, CH)], sbuf)
        pltpu.sync_copy(dsts[i].at[pl.ds(off, CH)], dbuf)

        def addoff(kk, _):
            sbuf[pl.ds(kk * 16, 16)] = sbuf[pl.ds(kk * 16, 16)] + coff
            return 0

        lax.fori_loop(0, CH // 16, addoff, 0, unroll=True)

    for i in range(P):
        pltpu.sync_copy(xflat_hbm.at[pl.ds(coff + row0, RPS)],
                        acc.at[pl.ds(row0, RPS)])

        @pl.when(s == NS - 1)
        def _():
            pltpu.sync_copy(xflat_hbm.at[pl.ds(coff + NS * RPS, RREM)],
                            acc.at[pl.ds(NS * RPS, RREM)])

        load_idx(i, 0, sidx_[0], didx_[0])
        plsc.subcore_barrier()

        def quad(k, _):
            for b4 in range(4):
                g = k * 4 + b4
                b = b4 % 2

                @pl.when(g >= 2)
                def _():
                    pltpu.make_async_copy(rows_[b], acc.at[didx_[b4]],
                                          ssem_[b]).wait()

                pltpu.async_copy(xflat_hbm.at[sidx_[b]], rows_[b], gsem_[b])

                @pl.when(g + 1 < NCH)
                def _():
                    load_idx(i, g + 1, sidx_[b ^ 1], didx_[(b4 + 1) % 4])

                pltpu.make_async_copy(xflat_hbm.at[sidx_[b]], rows_[b],
                                      gsem_[b]).wait()
                pltpu.async_copy(rows_[b], acc.at[didx_[b4]], ssem_[b],
                                 add=True)
            return 0

        lax.fori_loop(0, NCH // 4, quad, 0)

        pltpu.make_async_copy(rows_[0], acc.at[didx_[2]], ssem_[0]).wait()
        pltpu.make_async_copy(rows_[1], acc.at[didx_[3]], ssem_[1]).wait()

        plsc.subcore_barrier()
        pltpu.sync_copy(acc.at[pl.ds(row0, RPS)],
                        out_hbm.at[i, c, pl.ds(row0, RPS)])

        @pl.when(s == NS - 1)
        def _():
            pltpu.sync_copy(acc.at[pl.ds(NS * RPS, RREM)],
                            out_hbm.at[i, c, pl.ds(NS * RPS, RREM)])

        plsc.subcore_barrier()


@functools.cache
def _sc_segsum_fn():
    return functools.partial(
        pl.kernel,
        out_type=jax.ShapeDtypeStruct((P, NC, N, H), _f32),
        mesh=plsc.VectorSubcoreMesh(core_axis_name="c", subcore_axis_name="s",
                                    num_cores=NC, num_subcores=NS),
        scratch_types=(
            [pltpu.VMEM((CH,), jnp.int32)] * 6
            + [pltpu.VMEM((CH, H), _f32)] * 2
            + [pltpu.VMEM_SHARED((ACCR, H), _f32)]
            + [pltpu.SemaphoreType.DMA] * 4
        ),
    )(_sc_body)


def _sc_segsum(*args):
    return _sc_segsum_fn()(*args)


def _rot_coeffs(ee):
    r1 = ee[:, :H]
    r2 = ee[:, H:]
    nrm = jnp.sqrt(r1 * r1 + r2 * r2)
    nrm = jnp.maximum(nrm, 1e-12)
    cc = r1 / nrm
    ss = r2 / nrm
    a_ = cc
    b_ = -ss
    d_ = ss * cc
    e_ = cc - ss * ss
    out = []
    for path in PATH_LIST:
        m00 = jnp.ones((H,), _f32)
        m01 = jnp.zeros((H,), _f32)
        m10 = jnp.zeros((H,), _f32)
        m11 = jnp.ones((H,), _f32)
        for et in path:
            j = et - 1
            n00 = a_[j] * m00 + b_[j] * m10
            n01 = a_[j] * m01 + b_[j] * m11
            n10 = d_[j] * m00 + e_[j] * m10
            n11 = d_[j] * m01 + e_[j] * m11
            m00, m01, m10, m11 = n00, n01, n10, n11
        out.append((m00, m01, m10, m11))
    return out


def _dot_t(x, w):
    return lax.dot_general(x, w, (((1,), (1,)), ((), ())),
                           preferred_element_type=_f32)


def _k1_body(ee_ref, ap_ref, w0_ref, w1_ref, w2_ref, bg_ref,
             wa1_ref, ba1_ref, wa2_ref, z_ref, w_ref):
    coeffs = _rot_coeffs(ee_ref[...])
    wrefs = (w0_ref, w1_ref, w2_ref)
    wcols = []
    for i in range(P):
        m00, m01, m10, m11 = coeffs[i]
        u1 = ap_ref[i, 0]
        u2 = ap_ref[i, 1]
        rot1 = u1 * m00[None, :] + u2 * m01[None, :]
        rot2 = u1 * m10[None, :] + u2 * m11[None, :]
        wi = wrefs[i][...]
        g = _dot_t(rot1, wi[:, :H]) + _dot_t(rot2, wi[:, H:]) + bg_ref[i][None, :]
        z = jnp.where(g > 0, g, jnp.exp(jnp.minimum(g, 0.0)) - 1.0)
        z_ref[i] = z
        y = jnp.tanh(_dot_t(z, wa1_ref[...]) + ba1_ref[0][None, :])
        wcols.append(jnp.sum(y * wa2_ref[0][None, :], axis=1))
    w_ref[...] = jnp.stack(wcols, axis=1)


def _k1(ee, ap, w0, w1, w2, bg, wa1, ba1, wa2):
    grid = (N // RT,)
    return pl.pallas_call(
        _k1_body,
        grid=grid,
        in_specs=[
            pl.BlockSpec((P, D), lambda t: (0, 0)),
            pl.BlockSpec((P, NC, RT, H), lambda t: (0, 0, t, 0)),
            pl.BlockSpec((D, D), lambda t: (0, 0)),
            pl.BlockSpec((D, D), lambda t: (0, 0)),
            pl.BlockSpec((D, D), lambda t: (0, 0)),
            pl.BlockSpec((P, D), lambda t: (0, 0)),
            pl.BlockSpec((H, D), lambda t: (0, 0)),
            pl.BlockSpec((1, H), lambda t: (0, 0)),
            pl.BlockSpec((1, H), lambda t: (0, 0)),
        ],
        out_specs=[
            pl.BlockSpec((P, RT, D), lambda t: (0, t, 0)),
            pl.BlockSpec((RT, P), lambda t: (t, 0)),
        ],
        out_shape=[
            jax.ShapeDtypeStruct((P, N, D), _f32),
            jax.ShapeDtypeStruct((N, P), _f32),
        ],
    )(ee, ap, w0, w1, w2, bg, wa1, ba1, wa2)


def _k2_body(z_ref, w_ref, out_ref):
    wm = jnp.mean(w_ref[...], axis=0)
    wm = wm - jnp.max(wm)
    ew = jnp.exp(wm)
    beta = ew / jnp.sum(ew)
    out_ref[...] = (beta[0] * z_ref[0] + beta[1] * z_ref[1]
                    + beta[2] * z_ref[2])


def _k2(z, w):
    grid = (N // RT,)
    return pl.pallas_call(
        _k2_body,
        grid=grid,
        in_specs=[
            pl.BlockSpec((P, RT, D), lambda t: (0, t, 0)),
            pl.BlockSpec((N, P), lambda t: (0, 0)),
        ],
        out_specs=pl.BlockSpec((RT, D), lambda t: (t, 0)),
        out_shape=jax.ShapeDtypeStruct((N, D), _f32),
    )(z, w)


def kernel(node_emb, edge_emb, edge_index0, edge_index1, edge_index2,
           Wg0, bg0, Wg1, bg1, Wg2, bg2, Wa1, ba1, Wa2):
    xflat = node_emb.reshape(N, H, 2).transpose(2, 0, 1).reshape(NC * N, H)

    def _prep(ei):
        sp = jnp.pad(ei[0].reshape(NS, EPW), ((0, 0), (0, EPAD)))
        dp = jnp.pad(ei[1].reshape(NS, EPW), ((0, 0), (0, EPAD)),
                     constant_values=N)
        return sp.reshape(-1), dp.reshape(-1)

    s0, d0 = _prep(edge_index0)
    s1, d1 = _prep(edge_index1)
    s2, d2 = _prep(edge_index2)
    ap = _sc_segsum(xflat, s0, d0, s1, d1, s2, d2)
    bg = jnp.stack([bg0, bg1, bg2], axis=0)
    z, w = _k1(edge_emb, ap, Wg0, Wg1, Wg2, bg,
               Wa1, ba1.reshape(1, H), Wa2)
    return _k2(z, w)


# speedup vs baseline: 1.5550x; 1.5550x over previous
import functools

import jax
import jax.numpy as jnp
from jax import lax
from jax.experimental import pallas as pl
from jax.experimental.pallas import tpu as pltpu
from jax.experimental.pallas import tpu_sc as plsc

N = 10000
E = 160000
D = 256
H = D // 2
P = 3
NC = 2
NS = 16
EPW = E // NS
CH = 128
NFULL = EPW // CH
TAIL = EPW - NFULL * CH
RPS = 624
RREM = N - NS * RPS
RT = 1000
PATH_LIST = ((1,), (1, 2), (1, 2, 3))

_f32 = jnp.float32


def _sc_body(xflat_hbm, s0, d0, s1, d1, s2, d2, out_hbm,
             sidx0, sidx1, didx0, didx1, rows0, rows1,
             sidxt, didxt, rowst, acc, gsem, ss0, ss1):
    c = lax.axis_index("c")
    s = lax.axis_index("s")
    ebase = s * EPW
    row0 = s * RPS
    coff = c * N
    srcs = (s0, s1, s2)
    dsts = (d0, d1, d2)
    sidx_ = (sidx0, sidx1)
    didx_ = (didx0, didx1)
    rows_ = (rows0, rows1)
    ssem_ = (ss0, ss1)
    for i in range(P):
        pltpu.sync_copy(xflat_hbm.at[pl.ds(coff + row0, RPS)],
                        acc.at[pl.ds(row0, RPS)])

        @pl.when(s == NS - 1)
        def _():
            pltpu.sync_copy(xflat_hbm.at[pl.ds(coff + NS * RPS, RREM)],
                            acc.at[pl.ds(NS * RPS, RREM)])

        plsc.subcore_barrier()

        def chunk(k, _):
            for b in range(2):
                off = ebase + (k * 2 + b) * CH
                sidx = sidx_[b]
                didx = didx_[b]
                rows = rows_[b]

                @pl.when(k > 0)
                def _():
                    pltpu.make_async_copy(rows, acc.at[didx], ssem_[b]).wait()

                pltpu.async_copy(srcs[i].at[pl.ds(off, CH)], sidx, gsem)
                pltpu.async_copy(dsts[i].at[pl.ds(off, CH)], didx, gsem)
                pltpu.make_async_copy(srcs[i].at[pl.ds(off, CH)], sidx,
                                      gsem).wait()
                pltpu.make_async_copy(dsts[i].at[pl.ds(off, CH)], didx,
                                      gsem).wait()

                def addoff(kk, _):
                    sidx[pl.ds(kk * 16, 16)] = sidx[pl.ds(kk * 16, 16)] + coff
                    return 0

                lax.fori_loop(0, CH // 16, addoff, 0, unroll=True)
                pltpu.async_copy(xflat_hbm.at[sidx], rows, gsem).wait()
                pltpu.async_copy(rows, acc.at[didx], ssem_[b], add=True)
            return 0

        lax.fori_loop(0, NFULL // 2, chunk, 0)

        for b in range(2):
            pltpu.make_async_copy(rows_[b], acc.at[didx_[b]], ssem_[b]).wait()

        toff = ebase + NFULL * CH
        pltpu.sync_copy(srcs[i].at[pl.ds(toff, TAIL)], sidxt)
        pltpu.sync_copy(dsts[i].at[pl.ds(toff, TAIL)], didxt)
        sidxt[pl.ds(0, 16)] = sidxt[pl.ds(0, 16)] + coff
        pltpu.async_copy(xflat_hbm.at[sidxt], rowst, gsem).wait()
        pltpu.sync_copy(rowst, acc.at[didxt], add=True)

        plsc.subcore_barrier()
        pltpu.sync_copy(acc.at[pl.ds(row0, RPS)],
                        out_hbm.at[i, c, pl.ds(row0, RPS)])

        @pl.when(s == NS - 1)
        def _():
            pltpu.sync_copy(acc.at[pl.ds(NS * RPS, RREM)],
                            out_hbm.at[i, c, pl.ds(NS * RPS, RREM)])

        plsc.subcore_barrier()


@functools.cache
def _sc_segsum_fn():
    return functools.partial(
        pl.kernel,
        out_type=jax.ShapeDtypeStruct((P, NC, N, H), _f32),
        mesh=plsc.VectorSubcoreMesh(core_axis_name="c", subcore_axis_name="s",
                                    num_cores=NC, num_subcores=NS),
        scratch_types=[
            pltpu.VMEM((CH,), jnp.int32),
            pltpu.VMEM((CH,), jnp.int32),
            pltpu.VMEM((CH,), jnp.int32),
            pltpu.VMEM((CH,), jnp.int32),
            pltpu.VMEM((CH, H), _f32),
            pltpu.VMEM((CH, H), _f32),
            pltpu.VMEM((TAIL,), jnp.int32),
            pltpu.VMEM((TAIL,), jnp.int32),
            pltpu.VMEM((TAIL, H), _f32),
            pltpu.VMEM_SHARED((N, H), _f32),
            pltpu.SemaphoreType.DMA,
            pltpu.SemaphoreType.DMA,
            pltpu.SemaphoreType.DMA,
        ],
    )(_sc_body)


def _sc_segsum(*args):
    return _sc_segsum_fn()(*args)


def _rot_coeffs(ee):
    r1 = ee[:, :H]
    r2 = ee[:, H:]
    nrm = jnp.sqrt(r1 * r1 + r2 * r2)
    nrm = jnp.maximum(nrm, 1e-12)
    cc = r1 / nrm
    ss = r2 / nrm
    a_ = cc
    b_ = -ss
    d_ = ss * cc
    e_ = cc - ss * ss
    out = []
    for path in PATH_LIST:
        m00 = jnp.ones((H,), _f32)
        m01 = jnp.zeros((H,), _f32)
        m10 = jnp.zeros((H,), _f32)
        m11 = jnp.ones((H,), _f32)
        for et in path:
            j = et - 1
            n00 = a_[j] * m00 + b_[j] * m10
            n01 = a_[j] * m01 + b_[j] * m11
            n10 = d_[j] * m00 + e_[j] * m10
            n11 = d_[j] * m01 + e_[j] * m11
            m00, m01, m10, m11 = n00, n01, n10, n11
        out.append((m00, m01, m10, m11))
    return out


def _dot_t(x, w):
    return lax.dot_general(x, w, (((1,), (1,)), ((), ())),
                           preferred_element_type=_f32)


def _k1_body(ee_ref, ap_ref, w0_ref, w1_ref, w2_ref, bg_ref,
             wa1_ref, ba1_ref, wa2_ref, z_ref, w_ref):
    coeffs = _rot_coeffs(ee_ref[...])
    wrefs = (w0_ref, w1_ref, w2_ref)
    wcols = []
    for i in range(P):
        m00, m01, m10, m11 = coeffs[i]
        u1 = ap_ref[i, 0]
        u2 = ap_ref[i, 1]
        rot1 = u1 * m00[None, :] + u2 * m01[None, :]
        rot2 = u1 * m10[None, :] + u2 * m11[None, :]
        wi = wrefs[i][...]
        g = _dot_t(rot1, wi[:, :H]) + _dot_t(rot2, wi[:, H:]) + bg_ref[i][None, :]
        z = jnp.where(g > 0, g, jnp.exp(jnp.minimum(g, 0.0)) - 1.0)
        z_ref[i] = z
        y = jnp.tanh(_dot_t(z, wa1_ref[...]) + ba1_ref[0][None, :])
        wcols.append(jnp.sum(y * wa2_ref[0][None, :], axis=1))
    w_ref[...] = jnp.stack(wcols, axis=1)


def _k1(ee, ap, w0, w1, w2, bg, wa1, ba1, wa2):
    grid = (N // RT,)
    return pl.pallas_call(
        _k1_body,
        grid=grid,
        in_specs=[
            pl.BlockSpec((P, D), lambda t: (0, 0)),
            pl.BlockSpec((P, NC, RT, H), lambda t: (0, 0, t, 0)),
            pl.BlockSpec((D, D), lambda t: (0, 0)),
            pl.BlockSpec((D, D), lambda t: (0, 0)),
            pl.BlockSpec((D, D), lambda t: (0, 0)),
            pl.BlockSpec((P, D), lambda t: (0, 0)),
            pl.BlockSpec((H, D), lambda t: (0, 0)),
            pl.BlockSpec((1, H), lambda t: (0, 0)),
            pl.BlockSpec((1, H), lambda t: (0, 0)),
        ],
        out_specs=[
            pl.BlockSpec((P, RT, D), lambda t: (0, t, 0)),
            pl.BlockSpec((RT, P), lambda t: (t, 0)),
        ],
        out_shape=[
            jax.ShapeDtypeStruct((P, N, D), _f32),
            jax.ShapeDtypeStruct((N, P), _f32),
        ],
    )(ee, ap, w0, w1, w2, bg, wa1, ba1, wa2)


def _k2_body(z_ref, w_ref, out_ref):
    wm = jnp.mean(w_ref[...], axis=0)
    wm = wm - jnp.max(wm)
    ew = jnp.exp(wm)
    beta = ew / jnp.sum(ew)
    out_ref[...] = (beta[0] * z_ref[0] + beta[1] * z_ref[1]
                    + beta[2] * z_ref[2])


def _k2(z, w):
    grid = (N // RT,)
    return pl.pallas_call(
        _k2_body,
        grid=grid,
        in_specs=[
            pl.BlockSpec((P, RT, D), lambda t: (0, t, 0)),
            pl.BlockSpec((N, P), lambda t: (0, 0)),
        ],
        out_specs=pl.BlockSpec((RT, D), lambda t: (t, 0)),
        out_shape=jax.ShapeDtypeStruct((N, D), _f32),
    )(z, w)


def kernel(node_emb, edge_emb, edge_index0, edge_index1, edge_index2,
           Wg0, bg0, Wg1, bg1, Wg2, bg2, Wa1, ba1, Wa2):
    xflat = node_emb.reshape(N, H, 2).transpose(2, 0, 1).reshape(NC * N, H)
    ap = _sc_segsum(xflat,
                    edge_index0[0], edge_index0[1],
                    edge_index1[0], edge_index1[1],
                    edge_index2[0], edge_index2[1])
    bg = jnp.stack([bg0, bg1, bg2], axis=0)
    z, w = _k1(edge_emb, ap, Wg0, Wg1, Wg2, bg,
               Wa1, ba1.reshape(1, H), Wa2)
    return _k2(z, w)
